# baseline (speedup 1.0000x reference)
import functools

import jax
import jax.numpy as jnp
from jax import lax
from jax.experimental import pallas as pl
from jax.experimental.pallas import tpu as pltpu
from jax.experimental.pallas import tpu_sc as plsc

_B = 8
_NR = 1024
_N = _B * _NR
_K = 16
_KS = 16
_FN = 8
_HID = 256
_NC = 40
_RT = 4
_RW = _NR // _RT
_F3P = 32


def _knn_spline_body(pos_ref, kpos_ref, wspl_ref, idx_ref, rel_ref, f3d_ref):
    r = pl.program_id(1)
    b = pl.program_id(0)
    p = pos_ref[0]
    prow = pos_ref[0, pl.ds(r * _RW, _RW), :]
    sq_col = jnp.sum(p * p, axis=1)
    sq_row = jnp.sum(prow * prow, axis=1)
    dots = lax.dot_general(prow, p, (((1,), (1,)), ((), ())),
                           preferred_element_type=jnp.float32)
    d2 = sq_row[:, None] + sq_col[None, :] - 2.0 * dots

    bits = lax.bitcast_convert_type(d2, jnp.int32)
    mono = jnp.where(bits >= 0, bits, ~bits)
    col = lax.broadcasted_iota(jnp.int32, (_RW, _NR), 1)
    keys = (mono & jnp.int32(-1024)) | col

    kpos = kpos_ref[...]
    kpsq = jnp.sum(kpos * kpos, axis=1)
    wspl = wspl_ref[...]

    cidx = lax.broadcasted_iota(jnp.int32, (_FN, _FN * 3), 1)
    fidx = lax.broadcasted_iota(jnp.int32, (_FN, _FN * 3), 0)
    e_feat = (cidx // 3 == fidx).astype(jnp.float32)
    cidx3 = lax.broadcasted_iota(jnp.int32, (3, _FN * 3), 1)
    didx3 = lax.broadcasted_iota(jnp.int32, (3, _FN * 3), 0)
    e_dir = (cidx3 % 3 == didx3).astype(jnp.float32)

    acc = jnp.zeros((_RW, _FN * 3), dtype=jnp.float32)
    sentinel = jnp.int32(2**31 - 1)
    idx_cols = []
    for j in range(_K + 1):
        m = jnp.min(keys, axis=1, keepdims=True)
        hit = keys == m
        if j > 0:
            onehot = hit.astype(jnp.float32)
            nbr = lax.dot_general(onehot, p, (((1,), (0,)), ((), ())),
                                  preferred_element_type=jnp.float32)
            rel = nbr - prow
            rel_ref[0, j - 1, :, :] = rel
            idx_cols.append((m & jnp.int32(1023)) + b * _NR)
            r2 = jnp.sum(rel * rel, axis=1)
            rk = lax.dot_general(rel, kpos, (((1,), (1,)), ((), ())),
                                 preferred_element_type=jnp.float32)
            basis = jnp.exp(-(r2[:, None] - 2.0 * rk + kpsq[None, :]))
            feat = lax.dot_general(basis, wspl, (((1,), (0,)), ((), ())),
                                   preferred_element_type=jnp.float32)
            inv = 1.0 / (jnp.sqrt(r2) + 1e-8)
            dirn = rel * inv[:, None]
            f24 = lax.dot_general(feat, e_feat, (((1,), (0,)), ((), ())),
                                  preferred_element_type=jnp.float32)
            d24 = lax.dot_general(dirn, e_dir, (((1,), (0,)), ((), ())),
                                  preferred_element_type=jnp.float32)
            acc = acc + f24 * d24
        if j < _K:
            keys = jnp.where(hit, sentinel, keys)
    idx_ref[0] = jnp.swapaxes(jnp.concatenate(idx_cols, axis=1), 0, 1)
    f3d = jax.nn.sigmoid(acc / _K).astype(jnp.bfloat16)
    f3d_ref[0] = jnp.concatenate(
        [f3d, jnp.zeros((_RW, _F3P - _FN * 3), dtype=jnp.bfloat16)], axis=1)


def _dd_head_body(nf_ref, rel_ref, w1a_ref, w1b_ref, b1_ref, w2_ref, b2_ref,
                  wn1_ref, bn1_ref, wn2_ref, bn2_ref, out_ref):
    w1a = w1a_ref[...]
    w1b = w1b_ref[...]
    b1 = b1_ref[...]
    w2 = w2_ref[...]
    b2 = b2_ref[...]
    acc = jnp.zeros((_NR, _HID), dtype=jnp.float32)
    for k in range(_K):
        nfk = nf_ref[0, pl.ds(k * _NR, _NR), :]
        relk = rel_ref[0, k].astype(jnp.bfloat16)
        h = lax.dot_general(nfk, w1a, (((1,), (0,)), ((), ())),
                            preferred_element_type=jnp.float32)
        h = h + lax.dot_general(relk, w1b, (((1,), (0,)), ((), ())),
                                preferred_element_type=jnp.float32)
        h = jnp.maximum(h + b1, 0.0).astype(jnp.bfloat16)
        h = lax.dot_general(h, w2, (((1,), (0,)), ((), ())),
                            preferred_element_type=jnp.float32)
        h = jnp.maximum(h + b2, 0.0)
        acc = jnp.maximum(acc, h)
    ys = jnp.sum(acc, axis=0, keepdims=True) / _NR
    y1 = lax.dot_general(ys, wn1_ref[...], (((1,), (0,)), ((), ())),
                         preferred_element_type=jnp.float32) + bn1_ref[...]
    y1 = jnp.where(y1 > 0, y1, jnp.exp(jnp.minimum(y1, 0.0)) - 1.0)
    y2 = lax.dot_general(y1, wn2_ref[...], (((1,), (0,)), ((), ())),
                         preferred_element_type=jnp.float32) + bn2_ref[...]
    m = jnp.max(y2, axis=1, keepdims=True)
    lse = jnp.log(jnp.sum(jnp.exp(y2 - m), axis=1, keepdims=True)) + m
    out_ref[0] = y2 - lse


_SC_CHUNK = 512


@functools.cache
def _sc_gather():
    info = plsc.get_sparse_core_info()
    nw = info.num_cores * info.num_subcores
    per_w = (_N * _K) // nw

    def body(idx_hbm, table_hbm, out_hbm, idx_v, rows_v, sem):
        wid = lax.axis_index("s") * info.num_cores + lax.axis_index("c")
        for c in range(per_w // _SC_CHUNK):
            base = wid * per_w + c * _SC_CHUNK
            pltpu.sync_copy(idx_hbm.at[pl.ds(base, _SC_CHUNK)], idx_v)
            pltpu.async_copy(table_hbm.at[idx_v], rows_v, sem).wait()
            pltpu.sync_copy(rows_v, out_hbm.at[pl.ds(base, _SC_CHUNK)])

    return pl.kernel(
        body,
        out_type=jax.ShapeDtypeStruct((_N * _K, _F3P), jnp.bfloat16),
        mesh=plsc.VectorSubcoreMesh(core_axis_name="c", subcore_axis_name="s"),
        scratch_types=[
            pltpu.VMEM((_SC_CHUNK,), jnp.int32),
            pltpu.VMEM((_SC_CHUNK, _F3P), jnp.bfloat16),
            pltpu.SemaphoreType.DMA,
        ],
        compiler_params=pltpu.CompilerParams(use_tc_tiling_on_sc=False),
    )


def kernel(pos, batch, edge_index, kernel_pos, W_spline, W1, b1, W2, b2,
           Wn1, bn1, Wn2, bn2):
    del batch, edge_index
    pos3 = pos.reshape(_B, _NR, 3)
    idx, rel, f3d = pl.pallas_call(
        _knn_spline_body,
        grid=(_B, _RT),
        in_specs=[
            pl.BlockSpec((1, _NR, 3), lambda b, r: (b, 0, 0)),
            pl.BlockSpec((_KS, 3), lambda b, r: (0, 0)),
            pl.BlockSpec((_KS, _FN), lambda b, r: (0, 0)),
        ],
        out_specs=[
            pl.BlockSpec((1, _K, _RW), lambda b, r: (b, 0, r)),
            pl.BlockSpec((1, _K, _RW, 3), lambda b, r: (b, 0, r, 0)),
            pl.BlockSpec((1, _RW, _F3P), lambda b, r: (b, r, 0)),
        ],
        out_shape=[
            jax.ShapeDtypeStruct((_B, _K, _NR), jnp.int32),
            jax.ShapeDtypeStruct((_B, _K, _NR, 3), jnp.float32),
            jax.ShapeDtypeStruct((_B, _NR, _F3P), jnp.bfloat16),
        ],
    )(pos3, kernel_pos, W_spline)

    nf = _sc_gather()(idx.reshape(_N * _K), f3d.reshape(_N, _F3P))

    w1a = jnp.concatenate(
        [W1[: _FN * 3], jnp.zeros((_F3P - _FN * 3, _HID), jnp.float32)],
        axis=0).astype(jnp.bfloat16)
    w1b = W1[_FN * 3:].astype(jnp.bfloat16)
    w2 = W2.astype(jnp.bfloat16)

    out = pl.pallas_call(
        _dd_head_body,
        grid=(_B,),
        in_specs=[
            pl.BlockSpec((1, _K * _NR, _F3P), lambda b: (b, 0, 0)),
            pl.BlockSpec((1, _K, _NR, 3), lambda b: (b, 0, 0, 0)),
            pl.BlockSpec((_F3P, _HID), lambda b: (0, 0)),
            pl.BlockSpec((3, _HID), lambda b: (0, 0)),
            pl.BlockSpec((1, _HID), lambda b: (0, 0)),
            pl.BlockSpec((_HID, _HID), lambda b: (0, 0)),
            pl.BlockSpec((1, _HID), lambda b: (0, 0)),
            pl.BlockSpec((_HID, _HID), lambda b: (0, 0)),
            pl.BlockSpec((1, _HID), lambda b: (0, 0)),
            pl.BlockSpec((_HID, _NC), lambda b: (0, 0)),
            pl.BlockSpec((1, _NC), lambda b: (0, 0)),
        ],
        out_specs=pl.BlockSpec((1, 1, _NC), lambda b: (b, 0, 0)),
        out_shape=jax.ShapeDtypeStruct((_B, 1, _NC), jnp.float32),
    )(nf.reshape(_B, _K * _NR, _F3P), rel, w1a, w1b, b1.reshape(1, _HID),
      w2, b2.reshape(1, _HID), Wn1, bn1.reshape(1, _HID), Wn2,
      bn2.reshape(1, _NC))
    return out.reshape(_B, _NC)


# speedup vs baseline: 1.2352x; 1.2352x over previous
import functools

import jax
import jax.numpy as jnp
from jax import lax
from jax.experimental import pallas as pl
from jax.experimental.pallas import tpu as pltpu
from jax.experimental.pallas import tpu_sc as plsc

_B = 8
_NR = 1024
_N = _B * _NR
_K = 16
_KS = 16
_FN = 8
_HID = 256
_NC = 40
_RT = 4
_RW = _NR // _RT
_PP = 16
_F3P = 32


def _knn_body(pos_ref, idx_ref):
    r = pl.program_id(1)
    b = pl.program_id(0)
    p = pos_ref[0]
    prow = pos_ref[0, pl.ds(r * _RW, _RW), :]
    q = p * p
    ones = jnp.ones((1, 3), dtype=jnp.float32)
    sq_col = lax.dot_general(ones, q, (((1,), (1,)), ((), ())),
                             preferred_element_type=jnp.float32)
    sq_row = jnp.sum(prow * prow, axis=1, keepdims=True)
    dots = lax.dot_general(prow, p, (((1,), (1,)), ((), ())),
                           preferred_element_type=jnp.float32)
    d2 = sq_row + sq_col - 2.0 * dots

    bits = lax.bitcast_convert_type(d2, jnp.int32)
    mono = jnp.where(bits >= 0, bits, ~bits)
    col = lax.broadcasted_iota(jnp.int32, (_RW, _NR), 1)
    keys = (mono & jnp.int32(-1024)) | col

    sentinel = jnp.int32(2**31 - 1)
    idx_cols = []
    for j in range(_K + 1):
        m = jnp.min(keys, axis=1, keepdims=True)
        if j > 0:
            idx_cols.append((m & jnp.int32(1023)) + b * _NR)
        if j < _K:
            keys = jnp.where(keys == m, sentinel, keys)
    idx_ref[0] = jnp.swapaxes(jnp.concatenate(idx_cols, axis=1), 0, 1)


def _spline_body(nbr_ref, pos_ref, kpos_ref, wspl_ref, rel_ref, f3d_ref):
    p = pos_ref[0]
    kpos = kpos_ref[...]
    kpsq = jnp.swapaxes(jnp.sum(kpos * kpos, axis=1, keepdims=True), 0, 1)
    wspl = wspl_ref[...]

    cidx = lax.broadcasted_iota(jnp.int32, (_FN, _FN * 3), 1)
    fidx = lax.broadcasted_iota(jnp.int32, (_FN, _FN * 3), 0)
    e_feat = (cidx // 3 == fidx).astype(jnp.float32)
    cidx3 = lax.broadcasted_iota(jnp.int32, (3, _FN * 3), 1)
    didx3 = lax.broadcasted_iota(jnp.int32, (3, _FN * 3), 0)
    e_dir = (cidx3 % 3 == didx3).astype(jnp.float32)

    acc = jnp.zeros((_NR, _FN * 3), dtype=jnp.float32)
    for k in range(_K):
        nbrk = nbr_ref[0, pl.ds(k * _NR, _NR), :3]
        rel = nbrk - p
        rel_ref[0, k] = rel
        r2 = jnp.sum(rel * rel, axis=1, keepdims=True)
        rk = lax.dot_general(rel, kpos, (((1,), (1,)), ((), ())),
                             preferred_element_type=jnp.float32)
        basis = jnp.exp(-(r2 - 2.0 * rk + kpsq))
        feat = lax.dot_general(basis, wspl, (((1,), (0,)), ((), ())),
                               preferred_element_type=jnp.float32)
        inv = 1.0 / (jnp.sqrt(r2) + 1e-8)
        f24 = lax.dot_general(feat, e_feat, (((1,), (0,)), ((), ())),
                              preferred_element_type=jnp.float32)
        d24 = lax.dot_general(rel * inv, e_dir, (((1,), (0,)), ((), ())),
                              preferred_element_type=jnp.float32)
        acc = acc + f24 * d24
    f3d = jax.nn.sigmoid(acc / _K).astype(jnp.bfloat16)
    f3d_ref[0] = jnp.concatenate(
        [f3d, jnp.zeros((_NR, _F3P - _FN * 3), dtype=jnp.bfloat16)], axis=1)


def _dd_head_body(nf_ref, rel_ref, w1a_ref, w1b_ref, b1_ref, w2_ref, b2_ref,
                  wn1_ref, bn1_ref, wn2_ref, bn2_ref, out_ref):
    w1a = w1a_ref[...]
    w1b = w1b_ref[...]
    b1 = b1_ref[...]
    w2 = w2_ref[...]
    b2 = b2_ref[...]
    acc = jnp.zeros((_NR, _HID), dtype=jnp.float32)
    for k in range(_K):
        nfk = nf_ref[0, pl.ds(k * _NR, _NR), :].astype(jnp.float32)
        relk = rel_ref[0, k]
        h = lax.dot_general(nfk, w1a, (((1,), (0,)), ((), ())),
                            preferred_element_type=jnp.float32)
        h = h + lax.dot_general(relk, w1b, (((1,), (0,)), ((), ())),
                                preferred_element_type=jnp.float32)
        h = jnp.maximum(h + b1, 0.0)
        h = lax.dot_general(h, w2, (((1,), (0,)), ((), ())),
                            preferred_element_type=jnp.float32)
        h = jnp.maximum(h + b2, 0.0)
        acc = jnp.maximum(acc, h)
    ys = jnp.sum(acc, axis=0, keepdims=True) / _NR
    y1 = lax.dot_general(ys, wn1_ref[...], (((1,), (0,)), ((), ())),
                         preferred_element_type=jnp.float32) + bn1_ref[...]
    y1 = jnp.where(y1 > 0, y1, jnp.exp(jnp.minimum(y1, 0.0)) - 1.0)
    y2 = lax.dot_general(y1, wn2_ref[...], (((1,), (0,)), ((), ())),
                         preferred_element_type=jnp.float32) + bn2_ref[...]
    m = jnp.max(y2, axis=1, keepdims=True)
    lse = jnp.log(jnp.sum(jnp.exp(y2 - m), axis=1, keepdims=True)) + m
    out_ref[0] = y2 - lse


_SC_CHUNK = 512


@functools.cache
def _sc_gather(width, dtype_name):
    dtype = jnp.dtype(dtype_name)
    info = plsc.get_sparse_core_info()
    nw = info.num_cores * info.num_subcores
    per_w = (_N * _K) // nw

    def body(idx_hbm, table_hbm, out_hbm, idx_v, rows_v, sem):
        wid = lax.axis_index("s") * info.num_cores + lax.axis_index("c")
        for c in range(per_w // _SC_CHUNK):
            base = wid * per_w + c * _SC_CHUNK
            pltpu.sync_copy(idx_hbm.at[pl.ds(base, _SC_CHUNK)], idx_v)
            pltpu.async_copy(table_hbm.at[idx_v], rows_v, sem).wait()
            pltpu.sync_copy(rows_v, out_hbm.at[pl.ds(base, _SC_CHUNK)])

    return pl.kernel(
        body,
        out_type=jax.ShapeDtypeStruct((_N * _K, width), dtype),
        mesh=plsc.VectorSubcoreMesh(core_axis_name="c", subcore_axis_name="s"),
        scratch_types=[
            pltpu.VMEM((_SC_CHUNK,), jnp.int32),
            pltpu.VMEM((_SC_CHUNK, width), dtype),
            pltpu.SemaphoreType.DMA,
        ],
        compiler_params=pltpu.CompilerParams(use_tc_tiling_on_sc=False),
    )


def kernel(pos, batch, edge_index, kernel_pos, W_spline, W1, b1, W2, b2,
           Wn1, bn1, Wn2, bn2):
    del batch, edge_index
    pos3 = pos.reshape(_B, _NR, 3)
    idx = pl.pallas_call(
        _knn_body,
        grid=(_B, _RT),
        in_specs=[pl.BlockSpec((1, _NR, 3), lambda b, r: (b, 0, 0))],
        out_specs=pl.BlockSpec((1, _K, _RW), lambda b, r: (b, 0, r)),
        out_shape=jax.ShapeDtypeStruct((_B, _K, _NR), jnp.int32),
    )(pos3)

    pos_pad = jnp.concatenate(
        [pos, jnp.zeros((_N, _PP - 3), jnp.float32)], axis=1)
    nbr = _sc_gather(_PP, "float32")(idx.reshape(_N * _K), pos_pad)

    rel, f3d = pl.pallas_call(
        _spline_body,
        grid=(_B,),
        in_specs=[
            pl.BlockSpec((1, _K * _NR, _PP), lambda b: (b, 0, 0)),
            pl.BlockSpec((1, _NR, 3), lambda b: (b, 0, 0)),
            pl.BlockSpec((_KS, 3), lambda b: (0, 0)),
            pl.BlockSpec((_KS, _FN), lambda b: (0, 0)),
        ],
        out_specs=[
            pl.BlockSpec((1, _K, _NR, 3), lambda b: (b, 0, 0, 0)),
            pl.BlockSpec((1, _NR, _F3P), lambda b: (b, 0, 0)),
        ],
        out_shape=[
            jax.ShapeDtypeStruct((_B, _K, _NR, 3), jnp.float32),
            jax.ShapeDtypeStruct((_B, _NR, _F3P), jnp.bfloat16),
        ],
    )(nbr.reshape(_B, _K * _NR, _PP), pos3, kernel_pos, W_spline)

    nf = _sc_gather(_F3P, "bfloat16")(idx.reshape(_N * _K),
                                      f3d.reshape(_N, _F3P))

    w1a = jnp.concatenate(
        [W1[: _FN * 3], jnp.zeros((_F3P - _FN * 3, _HID), jnp.float32)],
        axis=0)
    w1b = W1[_FN * 3:]

    out = pl.pallas_call(
        _dd_head_body,
        grid=(_B,),
        in_specs=[
            pl.BlockSpec((1, _K * _NR, _F3P), lambda b: (b, 0, 0)),
            pl.BlockSpec((1, _K, _NR, 3), lambda b: (b, 0, 0, 0)),
            pl.BlockSpec((_F3P, _HID), lambda b: (0, 0)),
            pl.BlockSpec((3, _HID), lambda b: (0, 0)),
            pl.BlockSpec((1, _HID), lambda b: (0, 0)),
            pl.BlockSpec((_HID, _HID), lambda b: (0, 0)),
            pl.BlockSpec((1, _HID), lambda b: (0, 0)),
            pl.BlockSpec((_HID, _HID), lambda b: (0, 0)),
            pl.BlockSpec((1, _HID), lambda b: (0, 0)),
            pl.BlockSpec((_HID, _NC), lambda b: (0, 0)),
            pl.BlockSpec((1, _NC), lambda b: (0, 0)),
        ],
        out_specs=pl.BlockSpec((1, 1, _NC), lambda b: (b, 0, 0)),
        out_shape=jax.ShapeDtypeStruct((_B, 1, _NC), jnp.float32),
    )(nf.reshape(_B, _K * _NR, _F3P), rel, w1a, w1b, b1.reshape(1, _HID),
      W2, b2.reshape(1, _HID), Wn1, bn1.reshape(1, _HID), Wn2,
      bn2.reshape(1, _NC))
    return out.reshape(_B, _NC)
